# baseline (speedup 1.0000x reference)
import jax
import jax.numpy as jnp
from jax.experimental import pallas as pl
from jax.experimental.pallas import tpu as pltpu


def _make_body(bm, nblk):
    def _body(
        feat_ref,
        feat_a_ref,
        adj_ref,
        Wz_ref,
        W2_ref,
        z_ori_ref,
        z_aug_ref,
        x_ori_ref,
        x_aug_ref,
        s1_ref,
        s2_ref,
    ):
        p = pl.program_id(0)
        i = pl.program_id(1)
        h = z_ori_ref.shape[1]

        @pl.when((p == 0) & (i == 0))
        def _support():
            Wz = Wz_ref[...].astype(jnp.bfloat16)
            s1_ref[:, :h] = jnp.dot(
                feat_ref[...].astype(jnp.bfloat16),
                Wz,
                preferred_element_type=jnp.float32,
            ).astype(jnp.bfloat16)
            s1_ref[:, h:] = jnp.dot(
                feat_a_ref[...].astype(jnp.bfloat16),
                Wz,
                preferred_element_type=jnp.float32,
            ).astype(jnp.bfloat16)

        a = adj_ref[...].astype(jnp.bfloat16)

        @pl.when(p == 0)
        def _layer1():
            z = jnp.dot(a, s1_ref[...], preferred_element_type=jnp.float32)
            z_ori_ref[...] = z[:, :h]
            z_aug_ref[...] = z[:, h:]
            s2_ref[pl.ds(i * bm, bm), :] = jnp.dot(
                z.astype(jnp.bfloat16),
                W2_ref[...],
                preferred_element_type=jnp.float32,
            ).astype(jnp.bfloat16)

        @pl.when(p == 1)
        def _layer2():
            x = jnp.dot(a, s2_ref[...], preferred_element_type=jnp.float32)
            x_ori_ref[...] = x[:, :h]
            x_aug_ref[...] = x[:, h:]

    return _body


@jax.jit
def kernel(feat, feat_a, fadj, W_z, W_x):
    n, nfeat = feat.shape
    nhid = W_z.shape[1]
    nout = W_x.shape[1]
    f32 = jnp.float32

    bm = 200
    if n % bm != 0:
        for cand in (200, 100, 50, 25, 8, 5, 4, 2, 1):
            if n % cand == 0:
                bm = cand
                break
    nblk = n // bm

    zeros = jnp.zeros((nhid, nout), f32)
    W2 = jnp.block([[W_x, zeros], [zeros, W_x]]).astype(jnp.bfloat16)

    last = nblk - 1
    z_idx = lambda p, i: (jnp.where(p == 0, i, last), 0)
    x_idx = lambda p, i: (jnp.where(p == 1, i, 0), 0)

    z_ori, z_aug, xhat_ori, xhat_aug = pl.pallas_call(
        _make_body(bm, nblk),
        grid=(2, nblk),
        in_specs=[
            pl.BlockSpec((n, nfeat), lambda p, i: (0, 0)),
            pl.BlockSpec((n, nfeat), lambda p, i: (0, 0)),
            pl.BlockSpec((bm, n), lambda p, i: (i, 0)),
            pl.BlockSpec((nfeat, nhid), lambda p, i: (0, 0)),
            pl.BlockSpec((2 * nhid, 2 * nout), lambda p, i: (0, 0)),
        ],
        out_specs=[
            pl.BlockSpec((bm, nhid), z_idx),
            pl.BlockSpec((bm, nhid), z_idx),
            pl.BlockSpec((bm, nout), x_idx),
            pl.BlockSpec((bm, nout), x_idx),
        ],
        out_shape=[
            jax.ShapeDtypeStruct((n, nhid), f32),
            jax.ShapeDtypeStruct((n, nhid), f32),
            jax.ShapeDtypeStruct((n, nout), f32),
            jax.ShapeDtypeStruct((n, nout), f32),
        ],
        scratch_shapes=[
            pltpu.VMEM((n, 2 * nhid), jnp.bfloat16),
            pltpu.VMEM((n, 2 * nout), jnp.bfloat16),
        ],
        compiler_params=pltpu.CompilerParams(
            dimension_semantics=("arbitrary", "arbitrary"),
        ),
    )(feat, feat_a, fadj, W_z, W2)

    return (z_ori, z_aug, xhat_ori, xhat_aug)


# speedup vs baseline: 1.0790x; 1.0790x over previous
import jax
import jax.numpy as jnp
from jax.experimental import pallas as pl
from jax.experimental.pallas import tpu as pltpu


def _support1_body(feat_ref, feat_a_ref, W_ref, s1_ref):
    W = W_ref[...].astype(jnp.bfloat16)
    h = W.shape[1]
    s1_ref[:, :h] = jnp.dot(
        feat_ref[...].astype(jnp.bfloat16), W, preferred_element_type=jnp.float32
    ).astype(jnp.bfloat16)
    s1_ref[:, h:] = jnp.dot(
        feat_a_ref[...].astype(jnp.bfloat16), W, preferred_element_type=jnp.float32
    ).astype(jnp.bfloat16)


def _make_layers_body(bm):
    def _body(adj_ref, s1_ref, W2_ref, z_ori_ref, z_aug_ref, x_ori_ref,
              x_aug_ref, s2_ref):
        p = pl.program_id(0)
        i = pl.program_id(1)
        h = z_ori_ref.shape[1]
        a = adj_ref[...].astype(jnp.bfloat16)

        @pl.when(p == 0)
        def _layer1():
            z = jnp.dot(a, s1_ref[...], preferred_element_type=jnp.float32)
            z_ori_ref[...] = z[:, :h]
            z_aug_ref[...] = z[:, h:]
            s2_ref[pl.ds(i * bm, bm), :] = jnp.dot(
                z.astype(jnp.bfloat16),
                W2_ref[...],
                preferred_element_type=jnp.float32,
            ).astype(jnp.bfloat16)

        @pl.when(p == 1)
        def _layer2():
            x = jnp.dot(a, s2_ref[...], preferred_element_type=jnp.float32)
            x_ori_ref[...] = x[:, :h]
            x_aug_ref[...] = x[:, h:]

    return _body


@jax.jit
def kernel(feat, feat_a, fadj, W_z, W_x):
    n, nfeat = feat.shape
    nhid = W_z.shape[1]
    nout = W_x.shape[1]
    f32 = jnp.float32

    bm = 400
    if n % bm != 0:
        for cand in (200, 100, 50, 25, 8, 5, 4, 2, 1):
            if n % cand == 0:
                bm = cand
                break
    nblk = n // bm

    s1 = pl.pallas_call(
        _support1_body,
        out_shape=jax.ShapeDtypeStruct((n, 2 * nhid), jnp.bfloat16),
    )(feat, feat_a, W_z)

    zeros = jnp.zeros((nhid, nout), f32)
    W2 = jnp.block([[W_x, zeros], [zeros, W_x]]).astype(jnp.bfloat16)

    last = nblk - 1
    z_idx = lambda p, i: (jnp.where(p == 0, i, last), 0)
    x_idx = lambda p, i: (jnp.where(p == 1, i, 0), 0)

    z_ori, z_aug, xhat_ori, xhat_aug = pl.pallas_call(
        _make_layers_body(bm),
        grid=(2, nblk),
        in_specs=[
            pl.BlockSpec((bm, n), lambda p, i: (i, 0)),
            pl.BlockSpec((n, 2 * nhid), lambda p, i: (0, 0)),
            pl.BlockSpec((2 * nhid, 2 * nout), lambda p, i: (0, 0)),
        ],
        out_specs=[
            pl.BlockSpec((bm, nhid), z_idx),
            pl.BlockSpec((bm, nhid), z_idx),
            pl.BlockSpec((bm, nout), x_idx),
            pl.BlockSpec((bm, nout), x_idx),
        ],
        out_shape=[
            jax.ShapeDtypeStruct((n, nhid), f32),
            jax.ShapeDtypeStruct((n, nhid), f32),
            jax.ShapeDtypeStruct((n, nout), f32),
            jax.ShapeDtypeStruct((n, nout), f32),
        ],
        scratch_shapes=[
            pltpu.VMEM((n, 2 * nout), jnp.bfloat16),
        ],
        compiler_params=pltpu.CompilerParams(
            dimension_semantics=("arbitrary", "arbitrary"),
        ),
    )(fadj, s1, W2)

    return (z_ori, z_aug, xhat_ori, xhat_aug)
